# baseline (speedup 1.0000x reference)
import functools

import jax
import jax.numpy as jnp
from jax import lax
from jax.experimental import pallas as pl
from jax.experimental.pallas import tpu as pltpu
from jax.experimental.pallas import tpu_sc as plsc

NUM_ROWS = 16384
SEQ = 50
DIM = 32
TOTAL = NUM_ROWS * SEQ

_info = plsc.get_sparse_core_info()
NC, NS = _info.num_cores, _info.num_subcores
NW = NC * NS
PER_W = TOTAL // NW
CHUNK = 128
NCHUNK = PER_W // CHUNK
K = 4
NG = NCHUNK // K

_mesh = plsc.VectorSubcoreMesh(core_axis_name="c", subcore_axis_name="s")


@functools.partial(
    pl.kernel,
    mesh=_mesh,
    compiler_params=pltpu.CompilerParams(use_tc_tiling_on_sc=False),
    out_type=jax.ShapeDtypeStruct((TOTAL, DIM), jnp.float32),
    scratch_types=[
        pltpu.VMEM((NCHUNK, CHUNK), jnp.int32),
        pltpu.VMEM((K, CHUNK, DIM), jnp.float32),
        pltpu.VMEM((K, CHUNK, DIM), jnp.float32),
        pltpu.SemaphoreType.DMA,
        pltpu.SemaphoreType.DMA,
        pltpu.SemaphoreType.DMA,
        pltpu.SemaphoreType.DMA,
    ],
)
def _emb_lookup(idx_hbm, table_hbm, out_hbm, idx_v, rowsA, rowsB,
                gA, gB, oA, oB):
    wid = lax.axis_index("s") * NC + lax.axis_index("c")
    base = wid * PER_W
    rows = (rowsA, rowsB)
    gs = (gA, gB)
    os_ = (oA, oB)

    pltpu.sync_copy(idx_hbm.at[wid], idx_v)

    def out_ref(j):
        return out_hbm.at[pl.ds(base + j * CHUNK, CHUNK)]

    def fire_gathers(g, s):
        for b in range(K):
            pltpu.async_copy(table_hbm.at[idx_v.at[g * K + b]],
                             rows[s].at[b], gs[s])

    def drain_gathers(g, s):
        for b in range(K):
            pltpu.make_async_copy(table_hbm.at[idx_v.at[g * K + b]],
                                  rows[s].at[b], gs[s]).wait()

    def fire_writes(g, s):
        for b in range(K):
            pltpu.async_copy(rows[s].at[b], out_ref(g * K + b), os_[s])

    def drain_writes(g, s):
        for b in range(K):
            pltpu.make_async_copy(rows[s].at[b], out_ref(g * K + b),
                                  os_[s]).wait()

    fire_gathers(0, 0)
    fire_gathers(1, 1)
    drain_gathers(0, 0)
    fire_writes(0, 0)

    def body(p, _):
        for s in (1, 0):
            g = 2 * p + (1 if s == 1 else 2)
            drain_writes(g - 1, 1 - s)
            fire_gathers(g + 1, 1 - s)
            drain_gathers(g, s)
            fire_writes(g, s)
        return 0

    lax.fori_loop(0, (NG - 2) // 2, body, 0)

    drain_writes(NG - 2, 0)
    drain_gathers(NG - 1, 1)
    fire_writes(NG - 1, 1)
    drain_writes(NG - 1, 1)


def kernel(token_ids, E):
    idx = token_ids.astype(jnp.int32).reshape(NW, NCHUNK, CHUNK)
    out = _emb_lookup(idx, E)
    return out.reshape(NUM_ROWS, SEQ, DIM)


# speedup vs baseline: 1.0001x; 1.0001x over previous
import functools

import jax
import jax.numpy as jnp
from jax import lax
from jax.experimental import pallas as pl
from jax.experimental.pallas import tpu as pltpu
from jax.experimental.pallas import tpu_sc as plsc

NUM_ROWS = 16384
SEQ = 50
DIM = 32
TOTAL = NUM_ROWS * SEQ

_info = plsc.get_sparse_core_info()
NC, NS = _info.num_cores, _info.num_subcores
NW = NC * NS
PER_W = TOTAL // NW
CHUNK = 128
NCHUNK = PER_W // CHUNK
K = 5
NG = NCHUNK // K

_mesh = plsc.VectorSubcoreMesh(core_axis_name="c", subcore_axis_name="s")


@functools.partial(
    pl.kernel,
    mesh=_mesh,
    compiler_params=pltpu.CompilerParams(use_tc_tiling_on_sc=False),
    out_type=jax.ShapeDtypeStruct((TOTAL, DIM), jnp.float32),
    scratch_types=[
        pltpu.VMEM((NCHUNK, CHUNK), jnp.int32),
        pltpu.VMEM((K, CHUNK, DIM), jnp.float32),
        pltpu.VMEM((K, CHUNK, DIM), jnp.float32),
        pltpu.SemaphoreType.DMA,
        pltpu.SemaphoreType.DMA,
        pltpu.SemaphoreType.DMA,
        pltpu.SemaphoreType.DMA,
    ],
)
def _emb_lookup(idx_hbm, table_hbm, out_hbm, idx_v, rowsA, rowsB,
                gA, gB, oA, oB):
    wid = lax.axis_index("s") * NC + lax.axis_index("c")
    base = wid * PER_W
    rows = (rowsA, rowsB)
    gs = (gA, gB)
    os_ = (oA, oB)

    pltpu.sync_copy(idx_hbm.at[wid], idx_v)

    def out_ref(j):
        return out_hbm.at[pl.ds(base + j * CHUNK, CHUNK)]

    def fire_gathers(g, s):
        for b in range(K):
            pltpu.async_copy(table_hbm.at[idx_v.at[g * K + b]],
                             rows[s].at[b], gs[s])

    def drain_gathers(g, s):
        for b in range(K):
            pltpu.make_async_copy(table_hbm.at[idx_v.at[g * K + b]],
                                  rows[s].at[b], gs[s]).wait()

    def fire_writes(g, s):
        for b in range(K):
            pltpu.async_copy(rows[s].at[b], out_ref(g * K + b), os_[s])

    def drain_writes(g, s):
        for b in range(K):
            pltpu.make_async_copy(rows[s].at[b], out_ref(g * K + b),
                                  os_[s]).wait()

    fire_gathers(0, 0)
    fire_gathers(1, 1)
    drain_gathers(0, 0)
    fire_writes(0, 0)

    def body(p, _):
        for s in (1, 0):
            g = 2 * p + (1 if s == 1 else 2)
            drain_writes(g - 1, 1 - s)
            fire_gathers(g + 1, 1 - s)
            drain_gathers(g, s)
            fire_writes(g, s)
        return 0

    lax.fori_loop(0, (NG - 2) // 2, body, 0)

    drain_writes(NG - 2, 0)
    drain_gathers(NG - 1, 1)
    fire_writes(NG - 1, 1)
    drain_writes(NG - 1, 1)


def kernel(token_ids, E):
    idx = token_ids.astype(jnp.int32).reshape(NW, NCHUNK, CHUNK)
    out = _emb_lookup(idx, E)
    return out.reshape(NUM_ROWS, SEQ, DIM)
